# baseline (speedup 1.0000x reference)
import functools

import jax
import jax.numpy as jnp
from jax import lax
from jax.experimental import pallas as pl
from jax.experimental.pallas import tpu as pltpu
from jax.experimental.pallas import tpu_sc as plsc

NC = 2
NS = 16
NW = NC * NS

R = 4
SPLIT = 104


def _make_lookup(batch, hist, d):
    rows_per_w = batch // NW
    steps = rows_per_w // R
    outer_n = steps // 2
    fpw = rows_per_w * hist
    fps = R * hist
    n = batch * hist
    mesh = plsc.VectorSubcoreMesh(core_axis_name="c", subcore_axis_name="s")

    @functools.partial(
        pl.kernel,
        mesh=mesh,
        out_type=jax.ShapeDtypeStruct((n, 2, d), jnp.float32),
        scratch_types=[
            pltpu.VMEM((2, R, hist), jnp.int32),
            pltpu.VMEM((2, fps, d), jnp.float32),
            pltpu.SemaphoreType.DMA,
            pltpu.SemaphoreType.DMA,
        ],
        compiler_params=pltpu.CompilerParams(use_tc_tiling_on_sc=False),
    )
    def lookup(x_hbm, table_hbm, out_hbm, idx_v, rows_v, gsem, wsem):
        wid = lax.axis_index("s") * NC + lax.axis_index("c")
        base = wid * rows_per_w
        fbase = wid * fpw

        def fire(g, b):
            pltpu.sync_copy(x_hbm.at[pl.ds(base + g * R, R)], idx_v.at[b])
            for j in range(R):
                pltpu.async_copy(
                    table_hbm.at[idx_v.at[b, j, pl.ds(0, SPLIT)]],
                    rows_v.at[b, pl.ds(j * hist, SPLIT)],
                    gsem,
                )
                pltpu.async_copy(
                    table_hbm.at[idx_v.at[b, j, pl.ds(SPLIT, hist - SPLIT)]],
                    rows_v.at[b, pl.ds(j * hist + SPLIT, hist - SPLIT)],
                    gsem,
                )

        def wait_gathers(b):
            pltpu.make_async_copy(
                out_hbm.at[pl.ds(0, fps), 0], rows_v.at[b], gsem
            ).wait()

        def writeback(g, b):
            pltpu.async_copy(
                rows_v.at[b], out_hbm.at[pl.ds(fbase + g * fps, fps), 0], wsem
            )

        def wait_writeback(b):
            pltpu.make_async_copy(
                rows_v.at[b], out_hbm.at[pl.ds(0, fps), 0], wsem
            ).wait()

        fire(0, 0)

        def outer(i, carry):
            g = 2 * i

            @pl.when(i > 0)
            def _():
                wait_writeback(1)

            fire(g + 1, 1)
            wait_gathers(0)
            writeback(g, 0)

            @pl.when(i < outer_n - 1)
            def _():
                wait_writeback(0)
                fire(g + 2, 0)

            wait_gathers(1)
            writeback(g + 1, 1)
            return carry

        lax.fori_loop(0, outer_n, outer, 0)
        wait_writeback(0)
        wait_writeback(1)

    return lookup


def kernel(x, weight):
    b, h = x.shape
    v, d = weight.shape
    kout = _make_lookup(b, h, d)(x.astype(jnp.int32), weight)
    return kout[:, 0, :].reshape(b, h, d)


# speedup vs baseline: 2.5498x; 2.5498x over previous
import functools

import jax
import jax.numpy as jnp
from jax import lax
from jax.experimental import pallas as pl
from jax.experimental.pallas import tpu as pltpu
from jax.experimental.pallas import tpu_sc as plsc

NC = 2
NS = 16
NW = NC * NS

R = 4
SPLIT = 104


def _make_lookup(batch, hist, d):
    rows_per_w = batch // NW
    steps = rows_per_w // R
    outer_n = steps // 2
    mesh = plsc.VectorSubcoreMesh(core_axis_name="c", subcore_axis_name="s")

    @functools.partial(
        pl.kernel,
        mesh=mesh,
        out_type=jax.ShapeDtypeStruct((batch, hist, d), jnp.float32),
        scratch_types=[
            pltpu.VMEM((2, R, hist), jnp.int32),
            pltpu.VMEM((2, R, hist, d), jnp.float32),
            pltpu.SemaphoreType.DMA,
            pltpu.SemaphoreType.DMA,
        ],
        compiler_params=pltpu.CompilerParams(use_tc_tiling_on_sc=False),
    )
    def lookup(x_hbm, table_hbm, out_hbm, idx_v, rows_v, gsem, wsem):
        wid = lax.axis_index("s") * NC + lax.axis_index("c")
        base = wid * rows_per_w

        def fire(g, b):
            pltpu.sync_copy(x_hbm.at[pl.ds(base + g * R, R)], idx_v.at[b])
            for r in range(R):
                pltpu.async_copy(
                    table_hbm.at[idx_v.at[b, r, pl.ds(0, SPLIT)]],
                    rows_v.at[b, r, pl.ds(0, SPLIT)],
                    gsem,
                )
                pltpu.async_copy(
                    table_hbm.at[idx_v.at[b, r, pl.ds(SPLIT, hist - SPLIT)]],
                    rows_v.at[b, r, pl.ds(SPLIT, hist - SPLIT)],
                    gsem,
                )

        def wait_gathers(b):
            pltpu.make_async_copy(
                out_hbm.at[pl.ds(0, R)], rows_v.at[b], gsem
            ).wait()

        def writeback(g, b):
            pltpu.async_copy(
                rows_v.at[b], out_hbm.at[pl.ds(base + g * R, R)], wsem
            )

        def wait_writeback(b):
            pltpu.make_async_copy(
                rows_v.at[b], out_hbm.at[pl.ds(0, R)], wsem
            ).wait()

        fire(0, 0)

        def outer(i, carry):
            g = 2 * i

            @pl.when(i > 0)
            def _():
                wait_writeback(1)

            fire(g + 1, 1)
            wait_gathers(0)
            writeback(g, 0)

            @pl.when(i < outer_n - 1)
            def _():
                wait_writeback(0)
                fire(g + 2, 0)

            wait_gathers(1)
            writeback(g + 1, 1)
            return carry

        lax.fori_loop(0, outer_n, outer, 0)
        wait_writeback(0)
        wait_writeback(1)

    return lookup


def kernel(x, weight):
    b, h = x.shape
    v, d = weight.shape
    return _make_lookup(b, h, d)(x.astype(jnp.int32), weight)


# speedup vs baseline: 2.5722x; 1.0088x over previous
import functools

import jax
import jax.numpy as jnp
from jax import lax
from jax.experimental import pallas as pl
from jax.experimental.pallas import tpu as pltpu
from jax.experimental.pallas import tpu_sc as plsc

NC = 2
NS = 16
NW = NC * NS

R = 2
SPLIT = 104


def _make_lookup(batch, hist, d):
    rows_per_w = batch // NW
    steps = rows_per_w // R
    outer_n = steps // 2
    mesh = plsc.VectorSubcoreMesh(core_axis_name="c", subcore_axis_name="s")

    @functools.partial(
        pl.kernel,
        mesh=mesh,
        out_type=jax.ShapeDtypeStruct((batch, hist, d), jnp.float32),
        scratch_types=[
            pltpu.VMEM((rows_per_w, hist), jnp.int32),
            pltpu.VMEM((2, R, hist, d), jnp.float32),
            pltpu.SemaphoreType.DMA,
            pltpu.SemaphoreType.DMA,
        ],
        compiler_params=pltpu.CompilerParams(use_tc_tiling_on_sc=False),
    )
    def lookup(x_hbm, table_hbm, out_hbm, idx_v, rows_v, gsem, wsem):
        wid = lax.axis_index("s") * NC + lax.axis_index("c")
        base = wid * rows_per_w

        pltpu.sync_copy(x_hbm.at[pl.ds(base, rows_per_w)], idx_v)

        def fire(g, b):
            for r in range(R):
                pltpu.async_copy(
                    table_hbm.at[idx_v.at[g * R + r, pl.ds(0, SPLIT)]],
                    rows_v.at[b, r, pl.ds(0, SPLIT)],
                    gsem,
                )
                pltpu.async_copy(
                    table_hbm.at[idx_v.at[g * R + r, pl.ds(SPLIT, hist - SPLIT)]],
                    rows_v.at[b, r, pl.ds(SPLIT, hist - SPLIT)],
                    gsem,
                )

        def wait_gathers(b):
            pltpu.make_async_copy(
                out_hbm.at[pl.ds(0, R)], rows_v.at[b], gsem
            ).wait()

        def writeback(g, b):
            pltpu.async_copy(
                rows_v.at[b], out_hbm.at[pl.ds(base + g * R, R)], wsem
            )

        def wait_writeback(b):
            pltpu.make_async_copy(
                rows_v.at[b], out_hbm.at[pl.ds(0, R)], wsem
            ).wait()

        fire(0, 0)

        def outer(i, carry):
            g = 2 * i

            @pl.when(i > 0)
            def _():
                wait_writeback(1)

            fire(g + 1, 1)
            wait_gathers(0)
            writeback(g, 0)

            @pl.when(i < outer_n - 1)
            def _():
                wait_writeback(0)
                fire(g + 2, 0)

            wait_gathers(1)
            writeback(g + 1, 1)
            return carry

        lax.fori_loop(0, outer_n, outer, 0)
        wait_writeback(0)
        wait_writeback(1)

    return lookup


def kernel(x, weight):
    b, h = x.shape
    v, d = weight.shape
    return _make_lookup(b, h, d)(x.astype(jnp.int32), weight)
